# baseline (speedup 1.0000x reference)
import functools

import jax
import jax.numpy as jnp
from jax import lax
from jax.experimental import pallas as pl
from jax.experimental.pallas import tpu as pltpu
from jax.experimental.pallas import tpu_sc as plsc

EMB = 128
N_EDGES = 320000
ROWS01 = 21
ROWS2 = 3
T_PAD = 128

NW = 32
B_W = N_EDGES // NW
CHUNK = 400
TRIPS = B_W // CHUNK
LANES = 16


def _prep_body(ea_ref, w0_ref, w1_ref, w2_ref, c_ref, t_ref):
    c_ref[...] = ea_ref[0] * ROWS01 + ea_ref[1] * ROWS2 + ea_ref[2]
    r = lax.broadcasted_iota(jnp.int32, (T_PAD, 1), 0)
    i0 = r // ROWS01
    i1 = (r % ROWS01) // ROWS2
    i2 = r % ROWS2
    oh0 = (i0 == lax.broadcasted_iota(jnp.int32, (T_PAD, 6), 1)).astype(jnp.float32)
    oh1 = (i1 == lax.broadcasted_iota(jnp.int32, (T_PAD, 7), 1)).astype(jnp.float32)
    oh2 = (i2 == lax.broadcasted_iota(jnp.int32, (T_PAD, 3), 1)).astype(jnp.float32)
    t = jnp.dot(oh0, w0_ref[...], preferred_element_type=jnp.float32)
    t += jnp.dot(oh1, w1_ref[...], preferred_element_type=jnp.float32)
    t += jnp.dot(oh2, w2_ref[...], preferred_element_type=jnp.float32)
    t_ref[...] = t


def _sc_body(t_hbm, c_hbm, out_hbm, t_v, idx_v, ra, rb, ssem):
    bank = [ra, rb]
    wid = lax.axis_index("s") * 2 + lax.axis_index("c")
    pltpu.sync_copy(t_hbm, t_v)
    pltpu.sync_copy(c_hbm.at[wid], idx_v)
    lane = lax.broadcasted_iota(jnp.int32, (LANES,), 0)

    def fill(t, rows):
        def grp(g, carry):
            bases = idx_v[pl.ds(t * CHUNK + g * LANES, LANES)] * EMB
            for i in range(LANES):
                base = bases[i]
                ebase = (g * LANES + i) * EMB
                for j in range(0, EMB, LANES):
                    rows[pl.ds(ebase + j, LANES)] = t_v[pl.ds(base + j, LANES)]
            return carry

        lax.fori_loop(0, CHUNK // LANES, grp, 0)

    def out_slc(t):
        return out_hbm.at[pl.ds((wid * B_W + t * CHUNK) * EMB, CHUNK * EMB)]

    fill(0, bank[0])

    def trip(t, bk):
        rows, nrows = bank[bk], bank[1 - bk]

        @pl.when(t >= 1)
        def _():
            pltpu.make_async_copy(nrows, out_slc(t - 1), ssem).wait()

        pltpu.async_copy(rows, out_slc(t), ssem)

        @pl.when(t < TRIPS - 1)
        def _():
            fill(t + 1, nrows)

    def dbl(p, carry):
        trip(2 * p, 0)
        trip(2 * p + 1, 1)
        return carry

    lax.fori_loop(0, TRIPS // 2, dbl, 0)
    if TRIPS % 2:
        trip(TRIPS - 1, 0)
    pltpu.make_async_copy(bank[(TRIPS - 1) % 2], out_slc(TRIPS - 1), ssem).wait()


@jax.jit
def _run(ea_t, W0, W1, W2):
    c2d, table = pl.pallas_call(
        _prep_body,
        out_shape=(
            jax.ShapeDtypeStruct((N_EDGES // EMB, EMB), jnp.int32),
            jax.ShapeDtypeStruct((T_PAD, EMB), jnp.float32),
        ),
    )(ea_t, W0, W1, W2)

    mesh = plsc.VectorSubcoreMesh(core_axis_name="c", subcore_axis_name="s")
    sc = functools.partial(
        pl.kernel,
        out_type=jax.ShapeDtypeStruct((N_EDGES * EMB,), jnp.float32),
        mesh=mesh,
        compiler_params=pltpu.CompilerParams(needs_layout_passes=False),
        scratch_types=[
            pltpu.VMEM((T_PAD * EMB,), jnp.float32),
            pltpu.VMEM((B_W,), jnp.int32),
            pltpu.VMEM((CHUNK * EMB,), jnp.float32),
            pltpu.VMEM((CHUNK * EMB,), jnp.float32),
            pltpu.SemaphoreType.DMA,
        ],
    )(_sc_body)
    out = sc(table.reshape(T_PAD * EMB), c2d.reshape(NW, B_W))
    return out.reshape(N_EDGES, EMB)


def kernel(edge_attr, W0, W1, W2):
    ea_t = edge_attr.astype(jnp.int32).T.reshape(3, N_EDGES // EMB, EMB)
    return _run(ea_t, W0, W1, W2)


# speedup vs baseline: 2.4751x; 2.4751x over previous
import functools

import jax
import jax.numpy as jnp
from jax import lax
from jax.experimental import pallas as pl
from jax.experimental.pallas import tpu as pltpu
from jax.experimental.pallas import tpu_sc as plsc

EMB = 128
N_EDGES = 320000
ROWS01 = 21
ROWS2 = 3
T_PAD = 128

NW = 32
B_W = N_EDGES // NW
CHUNK = 400
TRIPS = B_W // CHUNK
LANES = 16


def _prep_body(ea_ref, w0_ref, w1_ref, w2_ref, c_ref, t_ref):
    c_ref[...] = ea_ref[0] * ROWS01 + ea_ref[1] * ROWS2 + ea_ref[2]
    r = lax.broadcasted_iota(jnp.int32, (T_PAD, 1), 0)
    i0 = r // ROWS01
    i1 = (r % ROWS01) // ROWS2
    i2 = r % ROWS2
    oh0 = (i0 == lax.broadcasted_iota(jnp.int32, (T_PAD, 6), 1)).astype(jnp.float32)
    oh1 = (i1 == lax.broadcasted_iota(jnp.int32, (T_PAD, 7), 1)).astype(jnp.float32)
    oh2 = (i2 == lax.broadcasted_iota(jnp.int32, (T_PAD, 3), 1)).astype(jnp.float32)
    t = jnp.dot(oh0, w0_ref[...], preferred_element_type=jnp.float32)
    t += jnp.dot(oh1, w1_ref[...], preferred_element_type=jnp.float32)
    t += jnp.dot(oh2, w2_ref[...], preferred_element_type=jnp.float32)
    t_ref[...] = t


def _sc_body(t_hbm, c_hbm, out_hbm, t_v, idx_v, ra, rb, ssem):
    bank = [ra, rb]
    wid = lax.axis_index("s") * 2 + lax.axis_index("c")
    pltpu.sync_copy(t_hbm, t_v)
    pltpu.sync_copy(c_hbm.at[wid], idx_v)
    lane = lax.broadcasted_iota(jnp.int32, (LANES,), 0)

    def fill(t, rows):
        def loads(base):
            return [t_v[pl.ds(base + j, LANES)] for j in range(0, EMB, LANES)]

        def stores(rows, ebase, vals):
            for j in range(0, EMB, LANES):
                rows[pl.ds(ebase + j, LANES)] = vals[j // LANES]

        def grp(g, carry):
            bases = idx_v[pl.ds(t * CHUNK + g * LANES, LANES)] * EMB
            ebase0 = g * LANES * EMB
            prev = loads(bases[0])
            for i in range(1, LANES):
                cur = loads(bases[i])
                stores(rows, ebase0 + (i - 1) * EMB, prev)
                prev = cur
            stores(rows, ebase0 + (LANES - 1) * EMB, prev)
            return carry

        lax.fori_loop(0, CHUNK // LANES, grp, 0)

    def out_slc(t):
        return out_hbm.at[pl.ds((wid * B_W + t * CHUNK) * EMB, CHUNK * EMB)]

    fill(0, bank[0])

    def trip(t, bk):
        rows, nrows = bank[bk], bank[1 - bk]

        @pl.when(t >= 1)
        def _():
            pltpu.make_async_copy(nrows, out_slc(t - 1), ssem).wait()

        pltpu.async_copy(rows, out_slc(t), ssem)

        @pl.when(t < TRIPS - 1)
        def _():
            fill(t + 1, nrows)

    def dbl(p, carry):
        trip(2 * p, 0)
        trip(2 * p + 1, 1)
        return carry

    lax.fori_loop(0, TRIPS // 2, dbl, 0)
    if TRIPS % 2:
        trip(TRIPS - 1, 0)
    pltpu.make_async_copy(bank[(TRIPS - 1) % 2], out_slc(TRIPS - 1), ssem).wait()


@jax.jit
def _run(ea_t, W0, W1, W2):
    c2d, table = pl.pallas_call(
        _prep_body,
        out_shape=(
            jax.ShapeDtypeStruct((N_EDGES // EMB, EMB), jnp.int32),
            jax.ShapeDtypeStruct((T_PAD, EMB), jnp.float32),
        ),
    )(ea_t, W0, W1, W2)

    mesh = plsc.VectorSubcoreMesh(core_axis_name="c", subcore_axis_name="s")
    sc = functools.partial(
        pl.kernel,
        out_type=jax.ShapeDtypeStruct((N_EDGES * EMB,), jnp.float32),
        mesh=mesh,
        compiler_params=pltpu.CompilerParams(needs_layout_passes=False),
        scratch_types=[
            pltpu.VMEM((T_PAD * EMB,), jnp.float32),
            pltpu.VMEM((B_W,), jnp.int32),
            pltpu.VMEM((CHUNK * EMB,), jnp.float32),
            pltpu.VMEM((CHUNK * EMB,), jnp.float32),
            pltpu.SemaphoreType.DMA,
        ],
    )(_sc_body)
    out = sc(table.reshape(T_PAD * EMB), c2d.reshape(NW, B_W))
    return out.reshape(N_EDGES, EMB)


def kernel(edge_attr, W0, W1, W2):
    ea_t = edge_attr.astype(jnp.int32).T.reshape(3, N_EDGES // EMB, EMB)
    return _run(ea_t, W0, W1, W2)


# speedup vs baseline: 2.4765x; 1.0005x over previous
import functools

import jax
import jax.numpy as jnp
from jax import lax
from jax.experimental import pallas as pl
from jax.experimental.pallas import tpu as pltpu
from jax.experimental.pallas import tpu_sc as plsc

EMB = 128
N_EDGES = 320000
ROWS01 = 21
ROWS2 = 3
T_PAD = 128

NW = 32
B_W = N_EDGES // NW
CHUNK = 400
TRIPS = B_W // CHUNK
LANES = 16


def _prep_body(ea_ref, w0_ref, w1_ref, w2_ref, c_ref, t_ref):
    c_ref[...] = ea_ref[0] * ROWS01 + ea_ref[1] * ROWS2 + ea_ref[2]
    r = lax.broadcasted_iota(jnp.int32, (T_PAD, 1), 0)
    i0 = r // ROWS01
    i1 = (r % ROWS01) // ROWS2
    i2 = r % ROWS2
    oh0 = (i0 == lax.broadcasted_iota(jnp.int32, (T_PAD, 6), 1)).astype(jnp.float32)
    oh1 = (i1 == lax.broadcasted_iota(jnp.int32, (T_PAD, 7), 1)).astype(jnp.float32)
    oh2 = (i2 == lax.broadcasted_iota(jnp.int32, (T_PAD, 3), 1)).astype(jnp.float32)
    t = jnp.dot(oh0, w0_ref[...], preferred_element_type=jnp.float32)
    t += jnp.dot(oh1, w1_ref[...], preferred_element_type=jnp.float32)
    t += jnp.dot(oh2, w2_ref[...], preferred_element_type=jnp.float32)
    t_ref[...] = t


def _sc_body(t_hbm, c_hbm, out_hbm, t_v, idx_v, ra, rb, ssem):
    bank = [ra, rb]
    wid = lax.axis_index("s") * 2 + lax.axis_index("c")
    pltpu.sync_copy(t_hbm, t_v)
    pltpu.sync_copy(c_hbm.at[wid], idx_v)
    lane = lax.broadcasted_iota(jnp.int32, (LANES,), 0)

    def fill(t, rows):
        def loads(base):
            return [t_v[pl.ds(base + j, LANES)] for j in range(0, EMB, LANES)]

        def stores(rows, ebase, vals):
            for j in range(0, EMB, LANES):
                rows[pl.ds(ebase + j, LANES)] = vals[j // LANES]

        def grp(g, carry):
            bvec = idx_v[pl.ds(t * CHUNK + g * LANES, LANES)] * EMB
            bases = [bvec[i] for i in range(LANES)]
            ebase0 = g * LANES * EMB
            prev = loads(bases[0])
            for i in range(1, LANES):
                cur = loads(bases[i])
                stores(rows, ebase0 + (i - 1) * EMB, prev)
                prev = cur
            stores(rows, ebase0 + (LANES - 1) * EMB, prev)
            return carry

        lax.fori_loop(0, CHUNK // LANES, grp, 0)

    def out_slc(t):
        return out_hbm.at[pl.ds((wid * B_W + t * CHUNK) * EMB, CHUNK * EMB)]

    fill(0, bank[0])

    def trip(t, bk):
        rows, nrows = bank[bk], bank[1 - bk]

        @pl.when(t >= 1)
        def _():
            pltpu.make_async_copy(nrows, out_slc(t - 1), ssem).wait()

        pltpu.async_copy(rows, out_slc(t), ssem)

        @pl.when(t < TRIPS - 1)
        def _():
            fill(t + 1, nrows)

    def dbl(p, carry):
        trip(2 * p, 0)
        trip(2 * p + 1, 1)
        return carry

    lax.fori_loop(0, TRIPS // 2, dbl, 0)
    if TRIPS % 2:
        trip(TRIPS - 1, 0)
    pltpu.make_async_copy(bank[(TRIPS - 1) % 2], out_slc(TRIPS - 1), ssem).wait()


@jax.jit
def _run(ea_t, W0, W1, W2):
    c2d, table = pl.pallas_call(
        _prep_body,
        out_shape=(
            jax.ShapeDtypeStruct((N_EDGES // EMB, EMB), jnp.int32),
            jax.ShapeDtypeStruct((T_PAD, EMB), jnp.float32),
        ),
    )(ea_t, W0, W1, W2)

    mesh = plsc.VectorSubcoreMesh(core_axis_name="c", subcore_axis_name="s")
    sc = functools.partial(
        pl.kernel,
        out_type=jax.ShapeDtypeStruct((N_EDGES * EMB,), jnp.float32),
        mesh=mesh,
        compiler_params=pltpu.CompilerParams(needs_layout_passes=False),
        scratch_types=[
            pltpu.VMEM((T_PAD * EMB,), jnp.float32),
            pltpu.VMEM((B_W,), jnp.int32),
            pltpu.VMEM((CHUNK * EMB,), jnp.float32),
            pltpu.VMEM((CHUNK * EMB,), jnp.float32),
            pltpu.SemaphoreType.DMA,
        ],
    )(_sc_body)
    out = sc(table.reshape(T_PAD * EMB), c2d.reshape(NW, B_W))
    return out.reshape(N_EDGES, EMB)


def kernel(edge_attr, W0, W1, W2):
    ea_t = edge_attr.astype(jnp.int32).T.reshape(3, N_EDGES // EMB, EMB)
    return _run(ea_t, W0, W1, W2)
